# baseline (speedup 1.0000x reference)
import functools

import jax
import jax.numpy as jnp
from jax import lax
from jax.experimental import pallas as pl
from jax.experimental.pallas import tpu as pltpu
from jax.experimental.pallas import tpu_sc as plsc

Q = 512
HW = 256
C = 128
ROWS = 16

NC = 2
NS = 16
L = 16
NW = NC * NS
QPW = Q // NW

_SC_MESH = plsc.VectorSubcoreMesh(
    core_axis_name="c", subcore_axis_name="s", num_cores=NC, num_subcores=NS)


def _lane_bcast(vec, lane):
    sel = jnp.full((L,), lane, jnp.int32)
    return vec.at[sel].get(mode="promise_in_bounds")


def _allreduce_min(v, lanes):
    for shift in (8, 4, 2, 1):
        perm = lanes ^ shift
        v = jnp.minimum(v, v.at[perm].get(mode="promise_in_bounds"))
    return v


@functools.partial(
    pl.kernel,
    out_type=jax.ShapeDtypeStruct((Q, C), jnp.float32),
    mesh=_SC_MESH,
    scratch_types=[
        pltpu.VMEM((QPW,), jnp.float32),
        pltpu.VMEM((QPW,), jnp.float32),
        pltpu.VMEM((HW,), jnp.float32),
        pltpu.VMEM((HW,), jnp.float32),
        pltpu.VMEM((QPW,), jnp.int32),
        pltpu.VMEM((QPW, C), jnp.float32),
        pltpu.SemaphoreType.DMA,
    ],
)
def _sc_nn_gather(qx_hbm, qy_hbm, gx_hbm, gy_hbm, tab_hbm, out_hbm,
                  qx_v, qy_v, gx_v, gy_v, idx_v, rows_v, sem):
    wid = lax.axis_index("s") * NC + lax.axis_index("c")
    base = wid * QPW
    pltpu.sync_copy(qx_hbm.at[pl.ds(base, QPW)], qx_v)
    pltpu.sync_copy(qy_hbm.at[pl.ds(base, QPW)], qy_v)
    pltpu.sync_copy(gx_hbm, gx_v)
    pltpu.sync_copy(gy_hbm, gy_v)
    qxv = qx_v[...]
    qyv = qy_v[...]
    lanes = lax.iota(jnp.int32, L)

    def per_query(qn, acc):
        qxb = _lane_bcast(qxv, qn)
        qyb = _lane_bcast(qyv, qn)
        bd = jnp.full((L,), jnp.inf, jnp.float32)
        bi = jnp.zeros((L,), jnp.int32)
        for ck in range(HW // L):
            gxc = gx_v[pl.ds(ck * L, L)]
            gyc = gy_v[pl.ds(ck * L, L)]
            dx = gxc - qxb
            dy = gyc - qyb
            d2 = dx * dx + dy * dy
            better = d2 < bd
            bi = jnp.where(better, lanes + ck * L, bi)
            bd = jnp.where(better, d2, bd)
        m = _allreduce_min(bd, lanes)
        cand = jnp.where(bd == m, bi, jnp.full((L,), 1 << 30, jnp.int32))
        best = _allreduce_min(cand, lanes)
        return jnp.where(lanes == qn, best, acc)

    idx_v[...] = lax.fori_loop(0, QPW, per_query,
                               jnp.zeros((L,), jnp.int32), unroll=True)
    pltpu.async_copy(tab_hbm.at[idx_v], rows_v, sem).wait()
    pltpu.sync_copy(rows_v, out_hbm.at[pl.ds(base, QPW)])


def _bcast_body(feat_ref, out_ref):
    out_ref[...] = jnp.broadcast_to(feat_ref[...][None], (ROWS, Q, C))


def kernel(grid_features, grid_coords, query_coords, N):
    gf = jnp.transpose(grid_features, (0, 2, 3, 1)).reshape(HW, C)
    gc = grid_coords.reshape(2, HW)
    feat = _sc_nn_gather(query_coords[:, 0], query_coords[:, 1],
                         gc[0], gc[1], gf)
    out = pl.pallas_call(
        _bcast_body,
        grid=(Q // ROWS,),
        in_specs=[pl.BlockSpec((Q, C), lambda i: (0, 0))],
        out_specs=pl.BlockSpec((ROWS, Q, C), lambda i: (i, 0, 0)),
        out_shape=jax.ShapeDtypeStruct((Q, Q, C), jnp.float32),
    )(feat)
    return out.reshape(1, Q, 1, Q, 1, C)


# speedup vs baseline: 1.3539x; 1.3539x over previous
import jax
import jax.numpy as jnp
from jax import lax
from jax.experimental import pallas as pl
from jax.experimental.pallas import tpu as pltpu

Q = 512
HW = 256
C = 128
W = 8


def _nn_broadcast_kernel(q_ref, gc_ref, gf_ref, out_ref, feat_ref, sem):
    q = q_ref[...]
    gc = gc_ref[...]
    qx = q[:, 0:1]
    qy = q[:, 1:2]
    gx = gc[0:1, :]
    gy = gc[1:2, :]
    dx = qx - gx
    dy = qy - gy
    d = jnp.sqrt(dx * dx + dy * dy)
    idx = jnp.argmin(d, axis=1)
    onehot = (idx[:, None] == jax.lax.broadcasted_iota(
        jnp.int32, (Q, HW), 1)).astype(jnp.float32)
    feat_ref[...] = jnp.dot(onehot, gf_ref[...],
                            preferred_element_type=jnp.float32,
                            precision=jax.lax.Precision.HIGHEST)

    def start(i, _):
        pltpu.make_async_copy(feat_ref, out_ref.at[i], sem).start()
        return 0

    def start_and_wait(i, _):
        pltpu.make_async_copy(feat_ref, out_ref.at[i], sem).start()
        pltpu.make_async_copy(feat_ref, out_ref.at[i - W], sem).wait()
        return 0

    def wait(i, _):
        pltpu.make_async_copy(feat_ref, out_ref.at[i], sem).wait()
        return 0

    lax.fori_loop(0, W, start, 0)
    lax.fori_loop(W, Q, start_and_wait, 0)
    lax.fori_loop(Q - W, Q, wait, 0)


def kernel(grid_features, grid_coords, query_coords, N):
    gf = jnp.transpose(grid_features, (0, 2, 3, 1)).reshape(HW, C)
    gc = grid_coords.reshape(2, HW)
    out = pl.pallas_call(
        _nn_broadcast_kernel,
        in_specs=[
            pl.BlockSpec((Q, 2), lambda: (0, 0)),
            pl.BlockSpec((2, HW), lambda: (0, 0)),
            pl.BlockSpec((HW, C), lambda: (0, 0)),
        ],
        out_specs=pl.BlockSpec(memory_space=pl.MemorySpace.ANY),
        out_shape=jax.ShapeDtypeStruct((Q, Q, C), jnp.float32),
        scratch_shapes=[pltpu.VMEM((Q, C), jnp.float32),
                        pltpu.SemaphoreType.DMA],
    )(query_coords, gc, gf)
    return out.reshape(1, Q, 1, Q, 1, C)


# speedup vs baseline: 1.4471x; 1.0688x over previous
import jax
import jax.numpy as jnp
from jax import lax
from jax.experimental import pallas as pl
from jax.experimental.pallas import tpu as pltpu

Q = 512
HW = 256
C = 128
W = 8
K = 4


def _nn_broadcast_kernel(q_ref, gc_ref, gf_ref, out_ref, feat_ref, sem):
    q = q_ref[...]
    gc = gc_ref[...]
    qx = q[:, 0:1]
    qy = q[:, 1:2]
    gx = gc[0:1, :]
    gy = gc[1:2, :]
    dx = qx - gx
    dy = qy - gy
    d = jnp.sqrt(dx * dx + dy * dy)
    idx = jnp.argmin(d, axis=1)
    onehot = (idx[:, None] == jax.lax.broadcasted_iota(
        jnp.int32, (Q, HW), 1)).astype(jnp.float32)
    feat = jnp.dot(onehot, gf_ref[...],
                   preferred_element_type=jnp.float32,
                   precision=jax.lax.Precision.HIGHEST)
    feat_ref[...] = jnp.broadcast_to(feat[None], (K, Q, C))
    ncopies = Q // K

    def start(i, _):
        pltpu.make_async_copy(feat_ref, out_ref.at[pl.ds(i * K, K)],
                              sem).start()
        return 0

    def start_and_wait(i, _):
        pltpu.make_async_copy(feat_ref, out_ref.at[pl.ds(i * K, K)],
                              sem).start()
        pltpu.make_async_copy(feat_ref, out_ref.at[pl.ds(0, K)], sem).wait()
        return 0

    def wait(i, _):
        pltpu.make_async_copy(feat_ref, out_ref.at[pl.ds(0, K)], sem).wait()
        return 0

    lax.fori_loop(0, W, start, 0)
    lax.fori_loop(W, ncopies, start_and_wait, 0)
    lax.fori_loop(0, W, wait, 0)


def kernel(grid_features, grid_coords, query_coords, N):
    gf = jnp.transpose(grid_features, (0, 2, 3, 1)).reshape(HW, C)
    gc = grid_coords.reshape(2, HW)
    out = pl.pallas_call(
        _nn_broadcast_kernel,
        in_specs=[
            pl.BlockSpec((Q, 2), lambda: (0, 0)),
            pl.BlockSpec((2, HW), lambda: (0, 0)),
            pl.BlockSpec((HW, C), lambda: (0, 0)),
        ],
        out_specs=pl.BlockSpec(memory_space=pl.MemorySpace.ANY),
        out_shape=jax.ShapeDtypeStruct((Q, Q, C), jnp.float32),
        scratch_shapes=[pltpu.VMEM((K, Q, C), jnp.float32),
                        pltpu.SemaphoreType.DMA],
    )(query_coords, gc, gf)
    return out.reshape(1, Q, 1, Q, 1, C)
